# baseline (speedup 1.0000x reference)
import jax
import jax.numpy as jnp
from jax import lax
from jax.experimental import pallas as pl
from jax.experimental.pallas import tpu as pltpu
from jax.experimental.pallas import tpu_sc as plsc

NUM_MOLECULES = 16384
NUM_ATOMS = 200
NUM_SPECIES = 8
HALF = NUM_ATOMS // 2

NC = 2
NS = 16
L = 16
NW = NC * NS
M_PER_W = NUM_MOLECULES // NW
CH = 128
NCHUNK = M_PER_W // CH
NPAIR = NCHUNK // 2
NGROUP = CH // L


def _body(speciesT_hbm, energies_hbm, ptab_hbm, out_hbm, sout_hbm,
          buf, ptab_v, ebuf, obuf,
          sem0, sem1, esem, ssem, wsem):
    wid = lax.axis_index("s") * NC + lax.axis_index("c")
    base = wid * M_PER_W
    HM = M_PER_W // 2

    QM = M_PER_W // 4
    qsems = (sem0, sem1, ssem, wsem)
    ecopy = pltpu.async_copy(
        energies_hbm.at[pl.ds(base, M_PER_W)], ebuf, esem)
    for q in range(4):
        pltpu.async_copy(
            speciesT_hbm.at[:, pl.ds(base + q * QM, QM)],
            buf.at[:, pl.ds(q * QM, QM)], qsems[q])
    pltpu.sync_copy(ptab_hbm, ptab_v)
    ecopy.wait()

    lanes = lax.iota(jnp.int32, L)
    GQ = QM // L

    def group_body(g, carry):
        for q in range(4):
            @pl.when(g == q * GQ)
            def _(q=q):
                pltpu.make_async_copy(
                    speciesT_hbm.at[:, pl.ds(base + q * QM, QM)],
                    buf.at[:, pl.ds(q * QM, QM)], qsems[q]).wait()
                pltpu.async_copy(
                    buf.at[:, pl.ds(q * QM, QM)],
                    sout_hbm.at[:, pl.ds(base + q * QM, QM)], qsems[q])

        off = g * L
        acc = [jnp.zeros((L,), jnp.float32) for _ in range(4)]
        for a in range(HALF):
            s_lo = buf[a, pl.ds(off, L)]
            s_hi = buf[a + HALF, pl.ds(off, L)]
            idx = (s_lo << 7) | (s_hi << 4) | lanes
            v = plsc.load_gather(ptab_v, [idx])
            acc[a % 4] = acc[a % 4] + v
        obuf[pl.ds(off, L)] = ((acc[0] + acc[1]) + (acc[2] + acc[3])
                               + ebuf[pl.ds(off, L)])
        return carry

    lax.fori_loop(0, M_PER_W // L, group_body, None)

    pltpu.sync_copy(obuf, out_hbm.at[pl.ds(base, M_PER_W)])
    for q in range(4):
        pltpu.make_async_copy(
            buf.at[:, pl.ds(q * QM, QM)],
            sout_hbm.at[:, pl.ds(base + q * QM, QM)], qsems[q]).wait()


@jax.jit
def _shifter(speciesT, energies, ptab):
    mesh = plsc.VectorSubcoreMesh(
        core_axis_name="c", subcore_axis_name="s",
        num_cores=NC, num_subcores=NS)
    run = pl.kernel(
        _body,
        out_type=(
            jax.ShapeDtypeStruct((NUM_MOLECULES,), jnp.float32),
            jax.ShapeDtypeStruct((NUM_ATOMS, NUM_MOLECULES), jnp.int32),
        ),
        mesh=mesh,
        scratch_types=[
            pltpu.VMEM((NUM_ATOMS, M_PER_W), jnp.int32),
            pltpu.VMEM((NUM_SPECIES * NUM_SPECIES * L,), jnp.float32),
            pltpu.VMEM((M_PER_W,), jnp.float32),
            pltpu.VMEM((M_PER_W,), jnp.float32),
            pltpu.SemaphoreType.DMA,
            pltpu.SemaphoreType.DMA,
            pltpu.SemaphoreType.DMA,
            pltpu.SemaphoreType.DMA,
            pltpu.SemaphoreType.DMA,
        ],
        compiler_params=pltpu.CompilerParams(
            use_tc_tiling_on_sc=True, needs_layout_passes=False),
    )
    return run(speciesT, energies, ptab)


def kernel(species, energies, self_energies):
    t = self_energies.astype(jnp.float32)
    ptab = jnp.broadcast_to(
        (t[:, None] + t[None, :])[:, :, None],
        (NUM_SPECIES, NUM_SPECIES, L)).reshape(-1)
    shifted, speciesT_out = _shifter(species.T, energies, ptab)
    return speciesT_out.T, shifted


# speedup vs baseline: 1.0354x; 1.0354x over previous
import jax
import jax.numpy as jnp
from jax import lax
from jax.experimental import pallas as pl
from jax.experimental.pallas import tpu as pltpu
from jax.experimental.pallas import tpu_sc as plsc

NUM_MOLECULES = 16384
NUM_ATOMS = 200
NUM_SPECIES = 8
HALF = NUM_ATOMS // 2

NC = 2
NS = 16
L = 16
NW = NC * NS
M_PER_W = NUM_MOLECULES // NW
CH = 128
NCHUNK = M_PER_W // CH
NPAIR = NCHUNK // 2
NGROUP = CH // L


def _body(speciesT_hbm, energies_hbm, ptab_hbm, out_hbm, sout_hbm,
          buf, ptab_v, ebuf, obuf,
          sem0, sem1, esem, ssem, wsem):
    wid = lax.axis_index("s") * NC + lax.axis_index("c")
    base = wid * M_PER_W
    HM = M_PER_W // 2

    ecopy = pltpu.async_copy(
        energies_hbm.at[pl.ds(base, M_PER_W)], ebuf, esem)
    fill0 = pltpu.async_copy(
        speciesT_hbm.at[:, pl.ds(base, HM)], buf.at[:, pl.ds(0, HM)], sem0)
    fill1 = pltpu.async_copy(
        speciesT_hbm.at[:, pl.ds(base + HM, HM)],
        buf.at[:, pl.ds(HM, HM)], sem1)
    pltpu.sync_copy(ptab_hbm, ptab_v)
    ecopy.wait()

    lanes = lax.iota(jnp.int32, L)

    def group_body(g, carry):
        @pl.when(g == HM // L)
        def _():
            fill1.wait()
            pltpu.async_copy(
                buf.at[:, pl.ds(HM, HM)],
                sout_hbm.at[:, pl.ds(base + HM, HM)], wsem)

        off = g * L
        acc0 = jnp.zeros((L,), jnp.float32)
        acc1 = jnp.zeros((L,), jnp.float32)
        for a in range(HALF):
            s_lo = buf[a, pl.ds(off, L)]
            s_hi = buf[a + HALF, pl.ds(off, L)]
            idx = (s_lo << 7) | (s_hi << 4) | lanes
            v = plsc.load_gather(ptab_v, [idx])
            if a % 2 == 0:
                acc0 = acc0 + v
            else:
                acc1 = acc1 + v
        obuf[pl.ds(off, L)] = (acc0 + acc1) + ebuf[pl.ds(off, L)]
        return carry

    fill0.wait()
    wb0 = pltpu.async_copy(
        buf.at[:, pl.ds(0, HM)], sout_hbm.at[:, pl.ds(base, HM)], ssem)

    lax.fori_loop(0, M_PER_W // L, group_body, None)

    pltpu.sync_copy(obuf, out_hbm.at[pl.ds(base, M_PER_W)])
    wb0.wait()
    pltpu.make_async_copy(
        buf.at[:, pl.ds(HM, HM)],
        sout_hbm.at[:, pl.ds(base + HM, HM)], wsem).wait()


@jax.jit
def _shifter(speciesT, energies, ptab):
    mesh = plsc.VectorSubcoreMesh(
        core_axis_name="c", subcore_axis_name="s",
        num_cores=NC, num_subcores=NS)
    run = pl.kernel(
        _body,
        out_type=(
            jax.ShapeDtypeStruct((NUM_MOLECULES,), jnp.float32),
            jax.ShapeDtypeStruct((NUM_ATOMS, NUM_MOLECULES), jnp.int32),
        ),
        mesh=mesh,
        scratch_types=[
            pltpu.VMEM((NUM_ATOMS, M_PER_W), jnp.int32),
            pltpu.VMEM((NUM_SPECIES * NUM_SPECIES * L,), jnp.float32),
            pltpu.VMEM((M_PER_W,), jnp.float32),
            pltpu.VMEM((M_PER_W,), jnp.float32),
            pltpu.SemaphoreType.DMA,
            pltpu.SemaphoreType.DMA,
            pltpu.SemaphoreType.DMA,
            pltpu.SemaphoreType.DMA,
            pltpu.SemaphoreType.DMA,
        ],
        compiler_params=pltpu.CompilerParams(
            use_tc_tiling_on_sc=True, needs_layout_passes=False),
    )
    return run(speciesT, energies, ptab)


def kernel(species, energies, self_energies):
    t = self_energies.astype(jnp.float32)
    ptab = jnp.broadcast_to(
        (t[:, None] + t[None, :])[:, :, None],
        (NUM_SPECIES, NUM_SPECIES, L)).reshape(-1)
    shifted, speciesT_out = _shifter(species.T, energies, ptab)
    return speciesT_out.T, shifted
